# baseline (speedup 1.0000x reference)
import functools

import jax
import jax.numpy as jnp
import numpy as np
from jax.experimental import pallas as pl
from jax.experimental.pallas import tpu as pltpu

_H = 8
_EPS = 1e-5
_THRESHOLD = 0.5

_VMEM = 58 * 2**20


def _params(grid_len):
    return pltpu.CompilerParams(
        dimension_semantics=("parallel",) + ("arbitrary",) * (grid_len - 1),
        vmem_limit_bytes=_VMEM,
        flags={"XLA_TPU_STORE_TO_LOAD_FORWARDING_WINDOW": 12288},
    )


def _ln_rows(t, g, b):
    mu = jnp.mean(t, axis=-1, keepdims=True)
    d = t - mu
    var = jnp.mean(d * d, axis=-1, keepdims=True)
    return d * jax.lax.rsqrt(var + _EPS) * g + b


def _xwT(x, w_ref):
    return jax.lax.dot_general(x, w_ref[...], (((1,), (1,)), ((), ())),
                               preferred_element_type=jnp.float32)


def _split_heads(qkv, dmodel, dh, q_ref, k_ref, v_ref):
    ones = jnp.ones((qkv.shape[0], dh), jnp.bfloat16)
    qs = jnp.bfloat16(1.0 / np.sqrt(dh))
    for h in range(_H):
        q_ref[0, h] = qkv[:, h * dh:(h + 1) * dh] * qs
        k_ref[0, h] = qkv[:, dmodel + h * dh:dmodel + (h + 1) * dh]
        v_ref[0, h] = jnp.concatenate(
            [qkv[:, 2 * dmodel + h * dh:2 * dmodel + (h + 1) * dh], ones],
            axis=-1)


def _qkv_body(x_ref, w_ref, b_ref, q_ref, k_ref, v_ref, *, dmodel, dh):
    qkv = _xwT(x_ref[0].astype(jnp.bfloat16), w_ref) + b_ref[...]
    _split_heads(qkv.astype(jnp.bfloat16), dmodel, dh, q_ref, k_ref, v_ref)


def _qkv_call(x, wT, b2):
    bsz, seq, dmodel = x.shape
    dh = dmodel // _H
    bm = min(512, seq)
    grid = (bsz, seq // bm)
    sds = jax.ShapeDtypeStruct((bsz, _H, seq, dh), jnp.bfloat16)
    vsds = jax.ShapeDtypeStruct((bsz, _H, seq, 2 * dh), jnp.bfloat16)
    return pl.pallas_call(
        functools.partial(_qkv_body, dmodel=dmodel, dh=dh),
        grid=grid,
        in_specs=[
            pl.BlockSpec((1, bm, dmodel), lambda bi, i: (bi, i, 0)),
            pl.BlockSpec((3 * dmodel, dmodel), lambda bi, i: (0, 0)),
            pl.BlockSpec((1, 3 * dmodel), lambda bi, i: (0, 0)),
        ],
        out_specs=[pl.BlockSpec((1, _H, bm, dh), lambda bi, i: (bi, 0, i, 0))] * 2
        + [pl.BlockSpec((1, _H, bm, 2 * dh), lambda bi, i: (bi, 0, i, 0))],
        out_shape=[sds, sds, vsds],
        compiler_params=_params(2),
        name=f"qkv_s{seq}",
    )(x, wT, b2)


def _fused_body(q_ref, k_ref, v_ref, x_ref, woT_ref, bo_ref, w1T_ref,
                b1_ref, w2T_ref, b2_ref, g1_ref, be1_ref, g2_ref, be2_ref,
                *rest, with_qkv, dmodel, dh):
    if with_qkv:
        wqkvT_ref, bqkv_ref, out_ref, qo_ref, ko_ref, vo_ref = rest
    else:
        (out_ref,) = rest
    os = []
    for h in range(_H):
        s = jax.lax.dot_general(q_ref[0, h], k_ref[0, h],
                                (((1,), (1,)), ((), ())),
                                preferred_element_type=jnp.float32)
        e = jnp.exp(jax.lax.clamp(-60.0, s, 60.0).astype(jnp.bfloat16))
        oe = jnp.dot(e, v_ref[0, h],
                     preferred_element_type=jnp.float32)
        os.append(oe[:, :dh] / oe[:, dh:])
    o_cat = jnp.concatenate(os, axis=-1).astype(jnp.bfloat16)
    t = x_ref[0] + _xwT(o_cat, woT_ref) + bo_ref[...]
    y1 = _ln_rows(t, g1_ref[...], be1_ref[...])
    hdn = jnp.maximum(_xwT(y1.astype(jnp.bfloat16), w1T_ref)
                      + b1_ref[...], 0.0)
    y2 = _ln_rows(y1 + _xwT(hdn.astype(jnp.bfloat16), w2T_ref)
                  + b2_ref[...],
                  g2_ref[...], be2_ref[...])
    out_ref[0] = y2
    if with_qkv:
        qkv = _xwT(y2.astype(jnp.bfloat16), wqkvT_ref) + bqkv_ref[...]
        _split_heads(qkv.astype(jnp.bfloat16), dmodel, dh,
                     qo_ref, ko_ref, vo_ref)


def _fused_call(qkv3, x, woT, bo2, w1T, b12, w2T, b22, g1, be1, g2, be2,
                nxt=None):
    q, k, v = qkv3
    bsz, seq, dmodel = x.shape
    dh = dmodel // _H
    dff = w1T.shape[0]
    bm = min(512, seq)
    grid = (bsz, seq // bm)
    with_qkv = nxt is not None
    full = lambda r, c: pl.BlockSpec((r, c), lambda bi, i: (0, 0))
    in_specs = [
        pl.BlockSpec((1, _H, bm, dh), lambda bi, i: (bi, 0, i, 0)),
        pl.BlockSpec((1, _H, seq, dh), lambda bi, i: (bi, 0, 0, 0)),
        pl.BlockSpec((1, _H, seq, 2 * dh), lambda bi, i: (bi, 0, 0, 0)),
        pl.BlockSpec((1, bm, dmodel), lambda bi, i: (bi, i, 0)),
        full(dmodel, dmodel), full(1, dmodel),
        full(dff, dmodel), full(1, dff),
        full(dmodel, dff), full(1, dmodel),
        full(1, dmodel), full(1, dmodel), full(1, dmodel), full(1, dmodel),
    ]
    out_specs = [pl.BlockSpec((1, bm, dmodel), lambda bi, i: (bi, i, 0))]
    out_shape = [jax.ShapeDtypeStruct((bsz, seq, dmodel), jnp.float32)]
    args = [q, k, v, x, woT, bo2, w1T, b12, w2T, b22, g1, be1, g2, be2]
    if with_qkv:
        wqkvT, bqkv2 = nxt
        in_specs += [full(3 * dmodel, dmodel), full(1, 3 * dmodel)]
        args += [wqkvT, bqkv2]
        qsds = jax.ShapeDtypeStruct((bsz, _H, seq, dh), jnp.bfloat16)
        vsds = jax.ShapeDtypeStruct((bsz, _H, seq, 2 * dh), jnp.bfloat16)
        out_specs += [pl.BlockSpec((1, _H, bm, dh),
                                   lambda bi, i: (bi, 0, i, 0))] * 2
        out_specs += [pl.BlockSpec((1, _H, bm, 2 * dh),
                                   lambda bi, i: (bi, 0, i, 0))]
        out_shape += [qsds, qsds, vsds]
    res = pl.pallas_call(
        functools.partial(_fused_body, with_qkv=with_qkv, dmodel=dmodel,
                          dh=dh),
        grid=grid,
        in_specs=in_specs,
        out_specs=out_specs,
        out_shape=out_shape,
        compiler_params=_params(2),
        name=f"layer_s{seq}{'_qkv' if with_qkv else ''}",
    )(*args)
    if with_qkv:
        return res[0], (res[1], res[2], res[3])
    return res[0]


def _chunk_body(bp_ref, x_ref, wqkvT_ref, bqkv_ref, c_ref, q_ref, k_ref,
                v_ref, *, seq, bc, dmodel, dh):
    bp = bp_ref[0]
    lane = jax.lax.broadcasted_iota(jnp.int32, (1, seq), 1)
    m = jnp.where(bp > _THRESHOLD, 1.0, 0.0)
    m = jnp.where(lane == 0, 0.0, m)
    s = m
    w = 1
    while w < seq:
        s = s + jnp.concatenate(
            [jnp.zeros((1, w), jnp.float32), s[:, :seq - w]], axis=-1)
        w *= 2
    seg = s - m
    c0 = pl.program_id(1) * bc
    cids = (jax.lax.broadcasted_iota(jnp.int32, (bc, seq), 0)
            + c0).astype(jnp.float32)
    p = jnp.where(cids == seg, 1.0, 0.0)
    sums = jnp.dot(p.astype(jnp.bfloat16), x_ref[0].astype(jnp.bfloat16),
                   preferred_element_type=jnp.float32)
    cnt = jnp.sum(p, axis=-1, keepdims=True)
    means = sums * (1.0 / jnp.maximum(cnt, 1.0))
    c_ref[0] = means
    qkv = _xwT(means.astype(jnp.bfloat16), wqkvT_ref) + bqkv_ref[...]
    _split_heads(qkv.astype(jnp.bfloat16), dmodel, dh, q_ref, k_ref, v_ref)


def _chunk_call(boundary_probs, x, wqkvT, bqkv2):
    bsz, seq, dmodel = x.shape
    dh = dmodel // _H
    half = seq // 2
    bc = min(512, half)
    grid = (bsz, half // bc)
    bp3 = boundary_probs.reshape(bsz, 1, seq)
    res = pl.pallas_call(
        functools.partial(_chunk_body, seq=seq, bc=bc, dmodel=dmodel, dh=dh),
        grid=grid,
        in_specs=[
            pl.BlockSpec((1, 1, seq), lambda bi, i: (bi, 0, 0)),
            pl.BlockSpec((1, seq, dmodel), lambda bi, i: (bi, 0, 0)),
            pl.BlockSpec((3 * dmodel, dmodel), lambda bi, i: (0, 0)),
            pl.BlockSpec((1, 3 * dmodel), lambda bi, i: (0, 0)),
        ],
        out_specs=[
            pl.BlockSpec((1, bc, dmodel), lambda bi, i: (bi, i, 0)),
            pl.BlockSpec((1, _H, bc, dh), lambda bi, i: (bi, 0, i, 0)),
            pl.BlockSpec((1, _H, bc, dh), lambda bi, i: (bi, 0, i, 0)),
            pl.BlockSpec((1, _H, bc, 2 * dh), lambda bi, i: (bi, 0, i, 0)),
        ],
        out_shape=[
            jax.ShapeDtypeStruct((bsz, half, dmodel), jnp.float32),
            jax.ShapeDtypeStruct((bsz, _H, half, dh), jnp.bfloat16),
            jax.ShapeDtypeStruct((bsz, _H, half, dh), jnp.bfloat16),
            jax.ShapeDtypeStruct((bsz, _H, half, 2 * dh), jnp.bfloat16),
        ],
        compiler_params=_params(2),
        name=f"chunk_s{seq}",
    )(bp3, x, wqkvT, bqkv2)
    return res[0], (res[1], res[2], res[3])


def kernel(x, boundary_probs, Wqkv, bqkv, Wo, bo, W1, b1, W2, b2,
           ln1_g, ln1_b, ln2_g, ln2_b):
    bf = jnp.bfloat16
    WqkvT, WoT, W1T, W2T = (Wqkv.astype(bf), Wo.astype(bf),
                            W1.astype(bf), W2.astype(bf))

    def post_args(l):
        return (WoT[l], bo[l][None], W1T[l], b1[l][None], W2T[l],
                b2[l][None], ln1_g[l][None], ln1_b[l][None],
                ln2_g[l][None], ln2_b[l][None])

    def nxt(l):
        return (WqkvT[l], bqkv[l][None])

    qkv = _qkv_call(x, WqkvT[0], bqkv[0][None])
    x1, qkv = _fused_call(qkv, x, *post_args(0), nxt=nxt(1))
    byte_encoded = _fused_call(qkv, x1, *post_args(1))

    chunks, qkv = _chunk_call(boundary_probs, byte_encoded, *nxt(2))

    x3, qkv = _fused_call(qkv, chunks, *post_args(2), nxt=nxt(3))
    return _fused_call(qkv, x3, *post_args(3))


# speedup vs baseline: 1.0197x; 1.0197x over previous
import functools

import jax
import jax.numpy as jnp
import numpy as np
from jax.experimental import pallas as pl
from jax.experimental.pallas import tpu as pltpu

_H = 8
_EPS = 1e-5
_THRESHOLD = 0.5

_VMEM = 58 * 2**20


def _params(grid_len):
    return pltpu.CompilerParams(
        dimension_semantics=("parallel",) + ("arbitrary",) * (grid_len - 1),
        vmem_limit_bytes=_VMEM,
        flags={"XLA_TPU_STORE_TO_LOAD_FORWARDING_WINDOW": 12288},
    )


def _ln_rows(t, g, b):
    mu = jnp.mean(t, axis=-1, keepdims=True)
    d = t - mu
    var = jnp.mean(d * d, axis=-1, keepdims=True)
    return d * jax.lax.rsqrt(var + _EPS) * g + b


def _xwT(x, w_ref):
    return jax.lax.dot_general(x, w_ref[...], (((1,), (1,)), ((), ())),
                               preferred_element_type=jnp.float32)


def _split_heads(qkv, dmodel, dh, q_ref, k_ref, v_ref):
    ones = jnp.ones((qkv.shape[0], dh), jnp.bfloat16)
    qs = jnp.bfloat16(1.0 / np.sqrt(dh))
    for h in range(_H):
        q_ref[0, h] = qkv[:, h * dh:(h + 1) * dh] * qs
        k_ref[0, h] = qkv[:, dmodel + h * dh:dmodel + (h + 1) * dh]
        v_ref[0, h] = jnp.concatenate(
            [qkv[:, 2 * dmodel + h * dh:2 * dmodel + (h + 1) * dh], ones],
            axis=-1)


def _qkv_body(x_ref, w_ref, b_ref, q_ref, k_ref, v_ref, *, dmodel, dh):
    qkv = _xwT(x_ref[0].astype(jnp.bfloat16), w_ref) + b_ref[...]
    _split_heads(qkv.astype(jnp.bfloat16), dmodel, dh, q_ref, k_ref, v_ref)


def _qkv_call(x, wT, b2):
    bsz, seq, dmodel = x.shape
    dh = dmodel // _H
    bm = min(1024, seq)
    grid = (bsz, seq // bm)
    sds = jax.ShapeDtypeStruct((bsz, _H, seq, dh), jnp.bfloat16)
    vsds = jax.ShapeDtypeStruct((bsz, _H, seq, 2 * dh), jnp.bfloat16)
    return pl.pallas_call(
        functools.partial(_qkv_body, dmodel=dmodel, dh=dh),
        grid=grid,
        in_specs=[
            pl.BlockSpec((1, bm, dmodel), lambda bi, i: (bi, i, 0)),
            pl.BlockSpec((3 * dmodel, dmodel), lambda bi, i: (0, 0)),
            pl.BlockSpec((1, 3 * dmodel), lambda bi, i: (0, 0)),
        ],
        out_specs=[pl.BlockSpec((1, _H, bm, dh), lambda bi, i: (bi, 0, i, 0))] * 2
        + [pl.BlockSpec((1, _H, bm, 2 * dh), lambda bi, i: (bi, 0, i, 0))],
        out_shape=[sds, sds, vsds],
        compiler_params=_params(2),
        name=f"qkv_s{seq}",
    )(x, wT, b2)


def _fused_body(q_ref, k_ref, v_ref, x_ref, woT_ref, bo_ref, w1T_ref,
                b1_ref, w2T_ref, b2_ref, g1_ref, be1_ref, g2_ref, be2_ref,
                *rest, with_qkv, dmodel, dh):
    if with_qkv:
        wqkvT_ref, bqkv_ref, out_ref, qo_ref, ko_ref, vo_ref = rest
    else:
        (out_ref,) = rest
    os = []
    for h in range(_H):
        s = jax.lax.dot_general(q_ref[0, h], k_ref[0, h],
                                (((1,), (1,)), ((), ())),
                                preferred_element_type=jnp.float32)
        e = jnp.exp(jax.lax.clamp(-60.0, s, 60.0).astype(jnp.bfloat16))
        oe = jnp.dot(e, v_ref[0, h],
                     preferred_element_type=jnp.float32)
        os.append(oe[:, :dh] / oe[:, dh:])
    o_cat = jnp.concatenate(os, axis=-1).astype(jnp.bfloat16)
    t = x_ref[0] + _xwT(o_cat, woT_ref) + bo_ref[...]
    y1 = _ln_rows(t, g1_ref[...], be1_ref[...])
    hdn = jnp.maximum(_xwT(y1.astype(jnp.bfloat16), w1T_ref)
                      + b1_ref[...], 0.0)
    y2 = _ln_rows(y1 + _xwT(hdn.astype(jnp.bfloat16), w2T_ref)
                  + b2_ref[...],
                  g2_ref[...], be2_ref[...])
    out_ref[0] = y2
    if with_qkv:
        qkv = _xwT(y2.astype(jnp.bfloat16), wqkvT_ref) + bqkv_ref[...]
        _split_heads(qkv.astype(jnp.bfloat16), dmodel, dh,
                     qo_ref, ko_ref, vo_ref)


def _fused_call(qkv3, x, woT, bo2, w1T, b12, w2T, b22, g1, be1, g2, be2,
                nxt=None):
    q, k, v = qkv3
    bsz, seq, dmodel = x.shape
    dh = dmodel // _H
    dff = w1T.shape[0]
    bm = 512 if seq > 1024 else min(1024, seq)
    grid = (bsz, seq // bm)
    with_qkv = nxt is not None
    full = lambda r, c: pl.BlockSpec((r, c), lambda bi, i: (0, 0))
    in_specs = [
        pl.BlockSpec((1, _H, bm, dh), lambda bi, i: (bi, 0, i, 0)),
        pl.BlockSpec((1, _H, seq, dh), lambda bi, i: (bi, 0, 0, 0)),
        pl.BlockSpec((1, _H, seq, 2 * dh), lambda bi, i: (bi, 0, 0, 0)),
        pl.BlockSpec((1, bm, dmodel), lambda bi, i: (bi, i, 0)),
        full(dmodel, dmodel), full(1, dmodel),
        full(dff, dmodel), full(1, dff),
        full(dmodel, dff), full(1, dmodel),
        full(1, dmodel), full(1, dmodel), full(1, dmodel), full(1, dmodel),
    ]
    out_specs = [pl.BlockSpec((1, bm, dmodel), lambda bi, i: (bi, i, 0))]
    out_shape = [jax.ShapeDtypeStruct((bsz, seq, dmodel), jnp.float32)]
    args = [q, k, v, x, woT, bo2, w1T, b12, w2T, b22, g1, be1, g2, be2]
    if with_qkv:
        wqkvT, bqkv2 = nxt
        in_specs += [full(3 * dmodel, dmodel), full(1, 3 * dmodel)]
        args += [wqkvT, bqkv2]
        qsds = jax.ShapeDtypeStruct((bsz, _H, seq, dh), jnp.bfloat16)
        vsds = jax.ShapeDtypeStruct((bsz, _H, seq, 2 * dh), jnp.bfloat16)
        out_specs += [pl.BlockSpec((1, _H, bm, dh),
                                   lambda bi, i: (bi, 0, i, 0))] * 2
        out_specs += [pl.BlockSpec((1, _H, bm, 2 * dh),
                                   lambda bi, i: (bi, 0, i, 0))]
        out_shape += [qsds, qsds, vsds]
    res = pl.pallas_call(
        functools.partial(_fused_body, with_qkv=with_qkv, dmodel=dmodel,
                          dh=dh),
        grid=grid,
        in_specs=in_specs,
        out_specs=out_specs,
        out_shape=out_shape,
        compiler_params=_params(2),
        name=f"layer_s{seq}{'_qkv' if with_qkv else ''}",
    )(*args)
    if with_qkv:
        return res[0], (res[1], res[2], res[3])
    return res[0]


def _chunk_body(bp_ref, x_ref, wqkvT_ref, bqkv_ref, c_ref, q_ref, k_ref,
                v_ref, *, seq, bc, dmodel, dh):
    bp = bp_ref[0]
    lane = jax.lax.broadcasted_iota(jnp.int32, (1, seq), 1)
    m = jnp.where(bp > _THRESHOLD, 1.0, 0.0)
    m = jnp.where(lane == 0, 0.0, m)
    s = m
    w = 1
    while w < seq:
        s = s + jnp.concatenate(
            [jnp.zeros((1, w), jnp.float32), s[:, :seq - w]], axis=-1)
        w *= 2
    seg = s - m
    c0 = pl.program_id(1) * bc
    cids = (jax.lax.broadcasted_iota(jnp.int32, (bc, seq), 0)
            + c0).astype(jnp.float32)
    p = jnp.where(cids == seg, 1.0, 0.0)
    sums = jnp.dot(p.astype(jnp.bfloat16), x_ref[0].astype(jnp.bfloat16),
                   preferred_element_type=jnp.float32)
    cnt = jnp.sum(p, axis=-1, keepdims=True)
    means = sums * (1.0 / jnp.maximum(cnt, 1.0))
    c_ref[0] = means
    qkv = _xwT(means.astype(jnp.bfloat16), wqkvT_ref) + bqkv_ref[...]
    _split_heads(qkv.astype(jnp.bfloat16), dmodel, dh, q_ref, k_ref, v_ref)


def _chunk_call(boundary_probs, x, wqkvT, bqkv2):
    bsz, seq, dmodel = x.shape
    dh = dmodel // _H
    half = seq // 2
    bc = min(512, half)
    grid = (bsz, half // bc)
    bp3 = boundary_probs.reshape(bsz, 1, seq)
    res = pl.pallas_call(
        functools.partial(_chunk_body, seq=seq, bc=bc, dmodel=dmodel, dh=dh),
        grid=grid,
        in_specs=[
            pl.BlockSpec((1, 1, seq), lambda bi, i: (bi, 0, 0)),
            pl.BlockSpec((1, seq, dmodel), lambda bi, i: (bi, 0, 0)),
            pl.BlockSpec((3 * dmodel, dmodel), lambda bi, i: (0, 0)),
            pl.BlockSpec((1, 3 * dmodel), lambda bi, i: (0, 0)),
        ],
        out_specs=[
            pl.BlockSpec((1, bc, dmodel), lambda bi, i: (bi, i, 0)),
            pl.BlockSpec((1, _H, bc, dh), lambda bi, i: (bi, 0, i, 0)),
            pl.BlockSpec((1, _H, bc, dh), lambda bi, i: (bi, 0, i, 0)),
            pl.BlockSpec((1, _H, bc, 2 * dh), lambda bi, i: (bi, 0, i, 0)),
        ],
        out_shape=[
            jax.ShapeDtypeStruct((bsz, half, dmodel), jnp.float32),
            jax.ShapeDtypeStruct((bsz, _H, half, dh), jnp.bfloat16),
            jax.ShapeDtypeStruct((bsz, _H, half, dh), jnp.bfloat16),
            jax.ShapeDtypeStruct((bsz, _H, half, 2 * dh), jnp.bfloat16),
        ],
        compiler_params=_params(2),
        name=f"chunk_s{seq}",
    )(bp3, x, wqkvT, bqkv2)
    return res[0], (res[1], res[2], res[3])


def kernel(x, boundary_probs, Wqkv, bqkv, Wo, bo, W1, b1, W2, b2,
           ln1_g, ln1_b, ln2_g, ln2_b):
    bf = jnp.bfloat16
    WqkvT, WoT, W1T, W2T = (Wqkv.astype(bf), Wo.astype(bf),
                            W1.astype(bf), W2.astype(bf))

    def post_args(l):
        return (WoT[l], bo[l][None], W1T[l], b1[l][None], W2T[l],
                b2[l][None], ln1_g[l][None], ln1_b[l][None],
                ln2_g[l][None], ln2_b[l][None])

    def nxt(l):
        return (WqkvT[l], bqkv[l][None])

    qkv = _qkv_call(x, WqkvT[0], bqkv[0][None])
    x1, qkv = _fused_call(qkv, x, *post_args(0), nxt=nxt(1))
    byte_encoded = _fused_call(qkv, x1, *post_args(1))

    chunks, qkv = _chunk_call(boundary_probs, byte_encoded, *nxt(2))

    x3, qkv = _fused_call(qkv, chunks, *post_args(2), nxt=nxt(3))
    return _fused_call(qkv, x3, *post_args(3))


# speedup vs baseline: 1.0231x; 1.0033x over previous
import functools

import jax
import jax.numpy as jnp
import numpy as np
from jax.experimental import pallas as pl
from jax.experimental.pallas import tpu as pltpu

_H = 8
_EPS = 1e-5
_THRESHOLD = 0.5

_VMEM = 58 * 2**20


def _params(grid_len):
    return pltpu.CompilerParams(
        dimension_semantics=("parallel",) + ("arbitrary",) * (grid_len - 1),
        vmem_limit_bytes=_VMEM,
        flags={"XLA_TPU_STORE_TO_LOAD_FORWARDING_WINDOW": 12288},
    )


def _ln_rows(t, g, b):
    mu = jnp.mean(t, axis=-1, keepdims=True)
    d = t - mu
    var = jnp.mean(d * d, axis=-1, keepdims=True)
    return d * jax.lax.rsqrt(var + _EPS) * g + b


def _xwT(x, w_ref):
    return jax.lax.dot_general(x, w_ref[...], (((1,), (1,)), ((), ())),
                               preferred_element_type=jnp.float32)


def _split_heads(qkv, dmodel, dh, q_ref, k_ref, v_ref):
    ones = jnp.ones((qkv.shape[0], dh), jnp.bfloat16)
    qs = jnp.bfloat16(1.0 / np.sqrt(dh))
    for h in range(_H):
        q_ref[0, h] = qkv[:, h * dh:(h + 1) * dh] * qs
        k_ref[0, h] = qkv[:, dmodel + h * dh:dmodel + (h + 1) * dh]
        v_ref[0, h] = jnp.concatenate(
            [qkv[:, 2 * dmodel + h * dh:2 * dmodel + (h + 1) * dh], ones],
            axis=-1)


def _qkv_body(x_ref, w_ref, b_ref, q_ref, k_ref, v_ref, *, dmodel, dh):
    qkv = (_xwT(x_ref[0].astype(jnp.bfloat16), w_ref).astype(jnp.bfloat16)
           + b_ref[...].astype(jnp.bfloat16))
    _split_heads(qkv, dmodel, dh, q_ref, k_ref, v_ref)


def _qkv_call(x, wT, b2):
    bsz, seq, dmodel = x.shape
    dh = dmodel // _H
    bm = min(1024, seq)
    grid = (bsz, seq // bm)
    sds = jax.ShapeDtypeStruct((bsz, _H, seq, dh), jnp.bfloat16)
    vsds = jax.ShapeDtypeStruct((bsz, _H, seq, 2 * dh), jnp.bfloat16)
    return pl.pallas_call(
        functools.partial(_qkv_body, dmodel=dmodel, dh=dh),
        grid=grid,
        in_specs=[
            pl.BlockSpec((1, bm, dmodel), lambda bi, i: (bi, i, 0)),
            pl.BlockSpec((3 * dmodel, dmodel), lambda bi, i: (0, 0)),
            pl.BlockSpec((1, 3 * dmodel), lambda bi, i: (0, 0)),
        ],
        out_specs=[pl.BlockSpec((1, _H, bm, dh), lambda bi, i: (bi, 0, i, 0))] * 2
        + [pl.BlockSpec((1, _H, bm, 2 * dh), lambda bi, i: (bi, 0, i, 0))],
        out_shape=[sds, sds, vsds],
        compiler_params=_params(2),
        name=f"qkv_s{seq}",
    )(x, wT, b2)


def _fused_body(q_ref, k_ref, v_ref, x_ref, woT_ref, bo_ref, w1T_ref,
                b1_ref, w2T_ref, b2_ref, g1_ref, be1_ref, g2_ref, be2_ref,
                *rest, with_qkv, dmodel, dh):
    if with_qkv:
        wqkvT_ref, bqkv_ref, out_ref, qo_ref, ko_ref, vo_ref = rest
    else:
        (out_ref,) = rest
    os = []
    for h in range(_H):
        s = jax.lax.dot_general(q_ref[0, h], k_ref[0, h],
                                (((1,), (1,)), ((), ())),
                                preferred_element_type=jnp.float32)
        e = jnp.exp(jax.lax.clamp(-60.0, s, 60.0).astype(jnp.bfloat16))
        oe = jnp.dot(e, v_ref[0, h],
                     preferred_element_type=jnp.float32)
        os.append(oe[:, :dh] / oe[:, dh:])
    o_cat = jnp.concatenate(os, axis=-1).astype(jnp.bfloat16)
    t = x_ref[0] + _xwT(o_cat, woT_ref) + bo_ref[...]
    y1 = _ln_rows(t, g1_ref[...], be1_ref[...])
    hdn = jnp.maximum(
        _xwT(y1.astype(jnp.bfloat16), w1T_ref).astype(jnp.bfloat16)
        + b1_ref[...].astype(jnp.bfloat16), jnp.bfloat16(0.0))
    y2 = _ln_rows(y1 + _xwT(hdn, w2T_ref) + b2_ref[...],
                  g2_ref[...], be2_ref[...])
    out_ref[0] = y2
    if with_qkv:
        qkv = (_xwT(y2.astype(jnp.bfloat16), wqkvT_ref).astype(jnp.bfloat16)
               + bqkv_ref[...].astype(jnp.bfloat16))
        _split_heads(qkv, dmodel, dh, qo_ref, ko_ref, vo_ref)


def _fused_call(qkv3, x, woT, bo2, w1T, b12, w2T, b22, g1, be1, g2, be2,
                nxt=None):
    q, k, v = qkv3
    bsz, seq, dmodel = x.shape
    dh = dmodel // _H
    dff = w1T.shape[0]
    bm = 512 if seq > 1024 else min(1024, seq)
    grid = (bsz, seq // bm)
    with_qkv = nxt is not None
    full = lambda r, c: pl.BlockSpec((r, c), lambda bi, i: (0, 0))
    in_specs = [
        pl.BlockSpec((1, _H, bm, dh), lambda bi, i: (bi, 0, i, 0)),
        pl.BlockSpec((1, _H, seq, dh), lambda bi, i: (bi, 0, 0, 0)),
        pl.BlockSpec((1, _H, seq, 2 * dh), lambda bi, i: (bi, 0, 0, 0)),
        pl.BlockSpec((1, bm, dmodel), lambda bi, i: (bi, i, 0)),
        full(dmodel, dmodel), full(1, dmodel),
        full(dff, dmodel), full(1, dff),
        full(dmodel, dff), full(1, dmodel),
        full(1, dmodel), full(1, dmodel), full(1, dmodel), full(1, dmodel),
    ]
    out_specs = [pl.BlockSpec((1, bm, dmodel), lambda bi, i: (bi, i, 0))]
    out_shape = [jax.ShapeDtypeStruct((bsz, seq, dmodel), jnp.float32)]
    args = [q, k, v, x, woT, bo2, w1T, b12, w2T, b22, g1, be1, g2, be2]
    if with_qkv:
        wqkvT, bqkv2 = nxt
        in_specs += [full(3 * dmodel, dmodel), full(1, 3 * dmodel)]
        args += [wqkvT, bqkv2]
        qsds = jax.ShapeDtypeStruct((bsz, _H, seq, dh), jnp.bfloat16)
        vsds = jax.ShapeDtypeStruct((bsz, _H, seq, 2 * dh), jnp.bfloat16)
        out_specs += [pl.BlockSpec((1, _H, bm, dh),
                                   lambda bi, i: (bi, 0, i, 0))] * 2
        out_specs += [pl.BlockSpec((1, _H, bm, 2 * dh),
                                   lambda bi, i: (bi, 0, i, 0))]
        out_shape += [qsds, qsds, vsds]
    res = pl.pallas_call(
        functools.partial(_fused_body, with_qkv=with_qkv, dmodel=dmodel,
                          dh=dh),
        grid=grid,
        in_specs=in_specs,
        out_specs=out_specs,
        out_shape=out_shape,
        compiler_params=_params(2),
        name=f"layer_s{seq}{'_qkv' if with_qkv else ''}",
    )(*args)
    if with_qkv:
        return res[0], (res[1], res[2], res[3])
    return res[0]


def _chunk_body(bp_ref, x_ref, wqkvT_ref, bqkv_ref, c_ref, q_ref, k_ref,
                v_ref, *, seq, bc, dmodel, dh):
    bp = bp_ref[0]
    lane = jax.lax.broadcasted_iota(jnp.int32, (1, seq), 1)
    m = jnp.where(bp > _THRESHOLD, 1.0, 0.0)
    m = jnp.where(lane == 0, 0.0, m)
    s = m
    w = 1
    while w < seq:
        s = s + jnp.concatenate(
            [jnp.zeros((1, w), jnp.float32), s[:, :seq - w]], axis=-1)
        w *= 2
    seg = s - m
    c0 = pl.program_id(1) * bc
    cids = (jax.lax.broadcasted_iota(jnp.int32, (bc, seq), 0)
            + c0).astype(jnp.float32)
    p = jnp.where(cids == seg, 1.0, 0.0)
    sums = jnp.dot(p.astype(jnp.bfloat16), x_ref[0].astype(jnp.bfloat16),
                   preferred_element_type=jnp.float32)
    cnt = jnp.sum(p, axis=-1, keepdims=True)
    means = sums * (1.0 / jnp.maximum(cnt, 1.0))
    c_ref[0] = means
    qkv = (_xwT(means.astype(jnp.bfloat16), wqkvT_ref).astype(jnp.bfloat16)
           + bqkv_ref[...].astype(jnp.bfloat16))
    _split_heads(qkv, dmodel, dh, q_ref, k_ref, v_ref)


def _chunk_call(boundary_probs, x, wqkvT, bqkv2):
    bsz, seq, dmodel = x.shape
    dh = dmodel // _H
    half = seq // 2
    bc = min(1024, half)
    grid = (bsz, half // bc)
    bp3 = boundary_probs.reshape(bsz, 1, seq)
    res = pl.pallas_call(
        functools.partial(_chunk_body, seq=seq, bc=bc, dmodel=dmodel, dh=dh),
        grid=grid,
        in_specs=[
            pl.BlockSpec((1, 1, seq), lambda bi, i: (bi, 0, 0)),
            pl.BlockSpec((1, seq, dmodel), lambda bi, i: (bi, 0, 0)),
            pl.BlockSpec((3 * dmodel, dmodel), lambda bi, i: (0, 0)),
            pl.BlockSpec((1, 3 * dmodel), lambda bi, i: (0, 0)),
        ],
        out_specs=[
            pl.BlockSpec((1, bc, dmodel), lambda bi, i: (bi, i, 0)),
            pl.BlockSpec((1, _H, bc, dh), lambda bi, i: (bi, 0, i, 0)),
            pl.BlockSpec((1, _H, bc, dh), lambda bi, i: (bi, 0, i, 0)),
            pl.BlockSpec((1, _H, bc, 2 * dh), lambda bi, i: (bi, 0, i, 0)),
        ],
        out_shape=[
            jax.ShapeDtypeStruct((bsz, half, dmodel), jnp.float32),
            jax.ShapeDtypeStruct((bsz, _H, half, dh), jnp.bfloat16),
            jax.ShapeDtypeStruct((bsz, _H, half, dh), jnp.bfloat16),
            jax.ShapeDtypeStruct((bsz, _H, half, 2 * dh), jnp.bfloat16),
        ],
        compiler_params=_params(2),
        name=f"chunk_s{seq}",
    )(bp3, x, wqkvT, bqkv2)
    return res[0], (res[1], res[2], res[3])


def kernel(x, boundary_probs, Wqkv, bqkv, Wo, bo, W1, b1, W2, b2,
           ln1_g, ln1_b, ln2_g, ln2_b):
    bf = jnp.bfloat16
    WqkvT, WoT, W1T, W2T = (Wqkv.astype(bf), Wo.astype(bf),
                            W1.astype(bf), W2.astype(bf))

    def post_args(l):
        return (WoT[l], bo[l][None], W1T[l], b1[l][None], W2T[l],
                b2[l][None], ln1_g[l][None], ln1_b[l][None],
                ln2_g[l][None], ln2_b[l][None])

    def nxt(l):
        return (WqkvT[l], bqkv[l][None])

    qkv = _qkv_call(x, WqkvT[0], bqkv[0][None])
    x1, qkv = _fused_call(qkv, x, *post_args(0), nxt=nxt(1))
    byte_encoded = _fused_call(qkv, x1, *post_args(1))

    chunks, qkv = _chunk_call(boundary_probs, byte_encoded, *nxt(2))

    x3, qkv = _fused_call(qkv, chunks, *post_args(2), nxt=nxt(3))
    return _fused_call(qkv, x3, *post_args(3))


# speedup vs baseline: 1.0424x; 1.0189x over previous
import functools

import jax
import jax.numpy as jnp
import numpy as np
from jax.experimental import pallas as pl
from jax.experimental.pallas import tpu as pltpu

_H = 8
_EPS = 1e-5
_THRESHOLD = 0.5

_VMEM = 58 * 2**20


def _params(grid_len):
    return pltpu.CompilerParams(
        dimension_semantics=("parallel",) + ("arbitrary",) * (grid_len - 1),
        vmem_limit_bytes=_VMEM,
        flags={"XLA_TPU_STORE_TO_LOAD_FORWARDING_WINDOW": 12288},
    )


def _ln_rows(t, g, b):
    mu = jnp.mean(t, axis=-1, keepdims=True)
    m2 = jnp.mean(t * t, axis=-1, keepdims=True)
    var = m2 - mu * mu
    return (t - mu) * jax.lax.rsqrt(var + _EPS) * g + b


def _xwT(x, w_ref):
    return jax.lax.dot_general(x, w_ref[...], (((1,), (1,)), ((), ())),
                               preferred_element_type=jnp.float32)


def _split_heads(qkv, dmodel, dh, q_ref, k_ref, v_ref):
    ones = jnp.ones((qkv.shape[0], dh), jnp.bfloat16)
    qs = jnp.bfloat16(1.0 / np.sqrt(dh))
    for h in range(_H):
        q_ref[0, h] = qkv[:, h * dh:(h + 1) * dh] * qs
        k_ref[0, h] = qkv[:, dmodel + h * dh:dmodel + (h + 1) * dh]
        v_ref[0, h] = jnp.concatenate(
            [qkv[:, 2 * dmodel + h * dh:2 * dmodel + (h + 1) * dh], ones],
            axis=-1)


def _qkv_body(x_ref, w_ref, b_ref, q_ref, k_ref, v_ref, *, dmodel, dh):
    qkv = (_xwT(x_ref[0].astype(jnp.bfloat16), w_ref).astype(jnp.bfloat16)
           + b_ref[...].astype(jnp.bfloat16))
    _split_heads(qkv, dmodel, dh, q_ref, k_ref, v_ref)


def _qkv_call(x, wT, b2):
    bsz, seq, dmodel = x.shape
    dh = dmodel // _H
    bm = min(1024, seq)
    grid = (bsz, seq // bm)
    sds = jax.ShapeDtypeStruct((bsz, _H, seq, dh), jnp.bfloat16)
    vsds = jax.ShapeDtypeStruct((bsz, _H, seq, 2 * dh), jnp.bfloat16)
    return pl.pallas_call(
        functools.partial(_qkv_body, dmodel=dmodel, dh=dh),
        grid=grid,
        in_specs=[
            pl.BlockSpec((1, bm, dmodel), lambda bi, i: (bi, i, 0)),
            pl.BlockSpec((3 * dmodel, dmodel), lambda bi, i: (0, 0)),
            pl.BlockSpec((1, 3 * dmodel), lambda bi, i: (0, 0)),
        ],
        out_specs=[pl.BlockSpec((1, _H, bm, dh), lambda bi, i: (bi, 0, i, 0))] * 2
        + [pl.BlockSpec((1, _H, bm, 2 * dh), lambda bi, i: (bi, 0, i, 0))],
        out_shape=[sds, sds, vsds],
        compiler_params=_params(2),
        name=f"qkv_s{seq}",
    )(x, wT, b2)


def _fused_body(q_ref, k_ref, v_ref, x_ref, woT_ref, bo_ref, w1T_ref,
                b1_ref, w2T_ref, b2_ref, g1_ref, be1_ref, g2_ref, be2_ref,
                *rest, with_qkv, dmodel, dh):
    if with_qkv:
        wqkvT_ref, bqkv_ref, out_ref, qo_ref, ko_ref, vo_ref = rest
    else:
        (out_ref,) = rest
    os = []
    for h in range(_H):
        s = jax.lax.dot_general(q_ref[0, h], k_ref[0, h],
                                (((1,), (1,)), ((), ())),
                                preferred_element_type=jnp.float32)
        e = jnp.exp(jnp.minimum(s.astype(jnp.bfloat16), jnp.bfloat16(80.0)))
        oe = jnp.dot(e, v_ref[0, h],
                     preferred_element_type=jnp.float32)
        os.append(oe[:, :dh] / oe[:, dh:])
    o_cat = jnp.concatenate(os, axis=-1).astype(jnp.bfloat16)
    t = x_ref[0] + _xwT(o_cat, woT_ref) + bo_ref[...]
    y1 = _ln_rows(t, g1_ref[...], be1_ref[...])
    hdn = jnp.maximum(
        _xwT(y1.astype(jnp.bfloat16), w1T_ref).astype(jnp.bfloat16)
        + b1_ref[...].astype(jnp.bfloat16), jnp.bfloat16(0.0))
    y2 = _ln_rows(y1 + _xwT(hdn, w2T_ref) + b2_ref[...],
                  g2_ref[...], be2_ref[...])
    out_ref[0] = y2
    if with_qkv:
        qkv = (_xwT(y2.astype(jnp.bfloat16), wqkvT_ref).astype(jnp.bfloat16)
               + bqkv_ref[...].astype(jnp.bfloat16))
        _split_heads(qkv, dmodel, dh, qo_ref, ko_ref, vo_ref)


def _fused_call(qkv3, x, woT, bo2, w1T, b12, w2T, b22, g1, be1, g2, be2,
                nxt=None):
    q, k, v = qkv3
    bsz, seq, dmodel = x.shape
    dh = dmodel // _H
    dff = w1T.shape[0]
    bm = 512 if seq > 1024 else min(1024, seq)
    grid = (bsz, seq // bm)
    with_qkv = nxt is not None
    full = lambda r, c: pl.BlockSpec((r, c), lambda bi, i: (0, 0))
    in_specs = [
        pl.BlockSpec((1, _H, bm, dh), lambda bi, i: (bi, 0, i, 0)),
        pl.BlockSpec((1, _H, seq, dh), lambda bi, i: (bi, 0, 0, 0)),
        pl.BlockSpec((1, _H, seq, 2 * dh), lambda bi, i: (bi, 0, 0, 0)),
        pl.BlockSpec((1, bm, dmodel), lambda bi, i: (bi, i, 0)),
        full(dmodel, dmodel), full(1, dmodel),
        full(dff, dmodel), full(1, dff),
        full(dmodel, dff), full(1, dmodel),
        full(1, dmodel), full(1, dmodel), full(1, dmodel), full(1, dmodel),
    ]
    out_specs = [pl.BlockSpec((1, bm, dmodel), lambda bi, i: (bi, i, 0))]
    out_shape = [jax.ShapeDtypeStruct((bsz, seq, dmodel), jnp.float32)]
    args = [q, k, v, x, woT, bo2, w1T, b12, w2T, b22, g1, be1, g2, be2]
    if with_qkv:
        wqkvT, bqkv2 = nxt
        in_specs += [full(3 * dmodel, dmodel), full(1, 3 * dmodel)]
        args += [wqkvT, bqkv2]
        qsds = jax.ShapeDtypeStruct((bsz, _H, seq, dh), jnp.bfloat16)
        vsds = jax.ShapeDtypeStruct((bsz, _H, seq, 2 * dh), jnp.bfloat16)
        out_specs += [pl.BlockSpec((1, _H, bm, dh),
                                   lambda bi, i: (bi, 0, i, 0))] * 2
        out_specs += [pl.BlockSpec((1, _H, bm, 2 * dh),
                                   lambda bi, i: (bi, 0, i, 0))]
        out_shape += [qsds, qsds, vsds]
    res = pl.pallas_call(
        functools.partial(_fused_body, with_qkv=with_qkv, dmodel=dmodel,
                          dh=dh),
        grid=grid,
        in_specs=in_specs,
        out_specs=out_specs,
        out_shape=out_shape,
        compiler_params=_params(2),
        name=f"layer_s{seq}{'_qkv' if with_qkv else ''}",
    )(*args)
    if with_qkv:
        return res[0], (res[1], res[2], res[3])
    return res[0]


def _chunk_body(bp_ref, x_ref, wqkvT_ref, bqkv_ref, c_ref, q_ref, k_ref,
                v_ref, *, seq, bc, dmodel, dh):
    bp = bp_ref[0]
    lane = jax.lax.broadcasted_iota(jnp.int32, (1, seq), 1)
    m = jnp.where(bp > _THRESHOLD, 1.0, 0.0)
    m = jnp.where(lane == 0, 0.0, m)
    s = m
    w = 1
    while w < seq:
        s = s + jnp.concatenate(
            [jnp.zeros((1, w), jnp.float32), s[:, :seq - w]], axis=-1)
        w *= 2
    seg = s - m
    c0 = pl.program_id(1) * bc
    cids = (jax.lax.broadcasted_iota(jnp.int32, (bc, seq), 0)
            + c0).astype(jnp.float32)
    p = jnp.where(cids == seg, 1.0, 0.0)
    sums = jnp.dot(p.astype(jnp.bfloat16), x_ref[0].astype(jnp.bfloat16),
                   preferred_element_type=jnp.float32)
    cnt = jnp.sum(p, axis=-1, keepdims=True)
    means = sums * (1.0 / jnp.maximum(cnt, 1.0))
    c_ref[0] = means
    qkv = (_xwT(means.astype(jnp.bfloat16), wqkvT_ref).astype(jnp.bfloat16)
           + bqkv_ref[...].astype(jnp.bfloat16))
    _split_heads(qkv, dmodel, dh, q_ref, k_ref, v_ref)


def _chunk_call(boundary_probs, x, wqkvT, bqkv2):
    bsz, seq, dmodel = x.shape
    dh = dmodel // _H
    half = seq // 2
    bc = min(1024, half)
    grid = (bsz, half // bc)
    bp3 = boundary_probs.reshape(bsz, 1, seq)
    res = pl.pallas_call(
        functools.partial(_chunk_body, seq=seq, bc=bc, dmodel=dmodel, dh=dh),
        grid=grid,
        in_specs=[
            pl.BlockSpec((1, 1, seq), lambda bi, i: (bi, 0, 0)),
            pl.BlockSpec((1, seq, dmodel), lambda bi, i: (bi, 0, 0)),
            pl.BlockSpec((3 * dmodel, dmodel), lambda bi, i: (0, 0)),
            pl.BlockSpec((1, 3 * dmodel), lambda bi, i: (0, 0)),
        ],
        out_specs=[
            pl.BlockSpec((1, bc, dmodel), lambda bi, i: (bi, i, 0)),
            pl.BlockSpec((1, _H, bc, dh), lambda bi, i: (bi, 0, i, 0)),
            pl.BlockSpec((1, _H, bc, dh), lambda bi, i: (bi, 0, i, 0)),
            pl.BlockSpec((1, _H, bc, 2 * dh), lambda bi, i: (bi, 0, i, 0)),
        ],
        out_shape=[
            jax.ShapeDtypeStruct((bsz, half, dmodel), jnp.float32),
            jax.ShapeDtypeStruct((bsz, _H, half, dh), jnp.bfloat16),
            jax.ShapeDtypeStruct((bsz, _H, half, dh), jnp.bfloat16),
            jax.ShapeDtypeStruct((bsz, _H, half, 2 * dh), jnp.bfloat16),
        ],
        compiler_params=_params(2),
        name=f"chunk_s{seq}",
    )(bp3, x, wqkvT, bqkv2)
    return res[0], (res[1], res[2], res[3])


def kernel(x, boundary_probs, Wqkv, bqkv, Wo, bo, W1, b1, W2, b2,
           ln1_g, ln1_b, ln2_g, ln2_b):
    bf = jnp.bfloat16
    WqkvT, WoT, W1T, W2T = (Wqkv.astype(bf), Wo.astype(bf),
                            W1.astype(bf), W2.astype(bf))

    def post_args(l):
        return (WoT[l], bo[l][None], W1T[l], b1[l][None], W2T[l],
                b2[l][None], ln1_g[l][None], ln1_b[l][None],
                ln2_g[l][None], ln2_b[l][None])

    def nxt(l):
        return (WqkvT[l], bqkv[l][None])

    qkv = _qkv_call(x, WqkvT[0], bqkv[0][None])
    x1, qkv = _fused_call(qkv, x, *post_args(0), nxt=nxt(1))
    byte_encoded = _fused_call(qkv, x1, *post_args(1))

    chunks, qkv = _chunk_call(boundary_probs, byte_encoded, *nxt(2))

    x3, qkv = _fused_call(qkv, chunks, *post_args(2), nxt=nxt(3))
    return _fused_call(qkv, x3, *post_args(3))
